# baseline (speedup 1.0000x reference)
import jax
import jax.numpy as jnp
from jax.experimental import pallas as pl
from jax.experimental.pallas import tpu as pltpu

D = 768
NH = 12
DK = 64
F = 3072
F2 = F // 2
NE = 16
PATCH = 16
IMG = 224
NC = 1000
NB = 4
S = (IMG // PATCH) ** 2 + 1
TT = NB * S
CAP = int(2 * 2.0 * TT / NE)
f32 = jnp.float32
bf16 = jnp.bfloat16


def _bdot(a, b):
    return jnp.dot(a.astype(bf16), b.astype(bf16), preferred_element_type=f32)


def _ln(x, g, b):
    mu = jnp.mean(x, axis=-1, keepdims=True)
    var = jnp.mean(jnp.square(x - mu), axis=-1, keepdims=True)
    return (x - mu) / jnp.sqrt(var + 1e-6) * g + b


def _pe_k(p_ref, w_ref, b_ref, o_ref):
    o_ref[...] = _bdot(p_ref[...], w_ref[...]) + b_ref[...]


def _route_body(x_ref, g_ref, b_ref, wg_ref, xn_s, cmb_s):
    x = x_ref[...]
    xn = _ln(x, g_ref[...], b_ref[...])
    xn_s[...] = xn.astype(bf16)
    logits = jnp.dot(xn, wg_ref[...], preferred_element_type=f32)
    gates = jax.nn.softmax(logits, axis=-1)
    ei = jax.lax.broadcasted_iota(jnp.int32, (TT, NE), 1)
    mx1 = jnp.max(gates, axis=-1, keepdims=True)
    i1 = jnp.min(jnp.where(gates == mx1, ei, NE), axis=-1, keepdims=True)
    m1 = (ei == i1).astype(f32)
    gm = gates * (1.0 - m1)
    mx2 = jnp.max(gm, axis=-1, keepdims=True)
    i2 = jnp.min(jnp.where(gm == mx2, ei, NE), axis=-1, keepdims=True)
    m2 = (ei == i2).astype(f32)
    rt = jax.lax.broadcasted_iota(jnp.int32, (TT, TT), 0)
    ct = jax.lax.broadcasted_iota(jnp.int32, (TT, TT), 1)
    tri = (ct < rt).astype(f32)
    pos1 = jnp.dot(tri, m1, preferred_element_type=f32)
    pos2 = jnp.dot(tri, m2, preferred_element_type=f32) + jnp.sum(
        m1, axis=0, keepdims=True)
    m1c = m1 * (pos1 < CAP)
    m2c = m2 * (pos2 < CAP)
    g1 = jnp.sum(gates * m1c, axis=-1, keepdims=True)
    g2 = jnp.sum(gates * m2c, axis=-1, keepdims=True)
    den = g1 + g2 + 1e-9
    g1 = g1 / den
    g2 = g2 / den
    loc1 = jnp.sum(pos1 * m1c, axis=-1, keepdims=True).astype(jnp.int32)
    loc2 = jnp.sum(pos2 * m2c, axis=-1, keepdims=True).astype(jnp.int32)
    ci = jax.lax.broadcasted_iota(jnp.int32, (TT, CAP), 1)
    oh1 = (ci == loc1).astype(f32) * g1
    oh2 = (ci == loc2).astype(f32) * g2
    for e in range(NE):
        cmb_s[e] = (m1c[:, e:e + 1] * oh1
                    + m2c[:, e:e + 1] * oh2).astype(bf16)


def _layer_body(h_ref, g1_ref, bg1_ref, wqkv_ref, bqkv_ref, wo_ref, bo_ref,
                g2_ref, bg2_ref, wg_ref, w1_ref, b1_ref, w2_ref, b2_ref,
                o_ref, w1b_ref, w2b_ref, xn_s, cmb_s):
    e = pl.program_id(0)
    ch = pl.program_id(1)

    @pl.when((e == 0) & (ch == 0))
    def _prologue():
        for b in range(NB):
            x = h_ref[b]
            xn = _ln(x, g1_ref[...], bg1_ref[...])
            qkv = _bdot(xn, wqkv_ref[...]) + bqkv_ref[...]
            heads = []
            for hh in range(NH):
                q = qkv[:, hh * DK:(hh + 1) * DK]
                k = qkv[:, D + hh * DK:D + (hh + 1) * DK]
                v = qkv[:, 2 * D + hh * DK:2 * D + (hh + 1) * DK]
                s = _bdot(q, k.T) * (1.0 / 8.0)
                p = jax.nn.softmax(s, axis=-1)
                heads.append(_bdot(p, v))
            o = jnp.concatenate(heads, axis=-1)
            o_ref[pl.ds(b * S, S), :] = x + _bdot(o, wo_ref[...]) + bo_ref[...]
        _route_body(o_ref, g2_ref, bg2_ref, wg_ref, xn_s, cmb_s)

    w1c = w1_ref[0].astype(bf16)
    w2c = w2_ref[0].astype(bf16)
    if w1b_ref is not None:
        w1b_ref[0] = w1c
        w2b_ref[0] = w2c
    c = cmb_s[e].astype(f32)
    dm = (c > 0.0).astype(bf16)
    ein = jax.lax.dot_general(dm, xn_s[...],
                              (((0,), (0,)), ((), ())),
                              preferred_element_type=f32)
    hh = jnp.dot(ein.astype(bf16), w1c,
                 preferred_element_type=f32) + b1_ref[0]
    hh = jax.nn.gelu(hh)
    eo = jnp.dot(hh.astype(bf16), w2c,
                 preferred_element_type=f32)
    eo = eo + jnp.where(ch == 0, 1.0, 0.0) * b2_ref[0]
    o_ref[...] = o_ref[...] + _bdot(c, eo)


def _layer_store_k(h_ref, g1_ref, bg1_ref, wqkv_ref, bqkv_ref, wo_ref,
                   bo_ref, g2_ref, bg2_ref, wg_ref, w1_ref, b1_ref, w2_ref,
                   b2_ref, o_ref, w1b_ref, w2b_ref, xn_s, cmb_s):
    _layer_body(h_ref, g1_ref, bg1_ref, wqkv_ref, bqkv_ref, wo_ref, bo_ref,
                g2_ref, bg2_ref, wg_ref, w1_ref, b1_ref, w2_ref, b2_ref,
                o_ref, w1b_ref, w2b_ref, xn_s, cmb_s)


def _layer_plain_k(h_ref, g1_ref, bg1_ref, wqkv_ref, bqkv_ref, wo_ref,
                   bo_ref, g2_ref, bg2_ref, wg_ref, w1_ref, b1_ref, w2_ref,
                   b2_ref, o_ref, xn_s, cmb_s):
    _layer_body(h_ref, g1_ref, bg1_ref, wqkv_ref, bqkv_ref, wo_ref, bo_ref,
                g2_ref, bg2_ref, wg_ref, w1_ref, b1_ref, w2_ref, b2_ref,
                o_ref, None, None, xn_s, cmb_s)


def _final_k(h_ref, g_ref, b_ref, wc_ref, bc_ref, o_ref):
    xn = _ln(h_ref[...], g_ref[...], b_ref[...])
    bi = jax.lax.broadcasted_iota(jnp.int32, (NB, TT), 0)
    ti = jax.lax.broadcasted_iota(jnp.int32, (NB, TT), 1)
    pool = ((ti >= bi * S) & (ti < bi * S + S)).astype(f32) * (1.0 / S)
    pooled = jnp.dot(pool, xn, preferred_element_type=f32)
    o_ref[...] = _bdot(pooled, wc_ref[...]) + bc_ref[...]


def kernel(x, Wp, bp, cls_tok, pos, Wqkv, bqkv, Wo, bo, Wg, W1, b1, W2, b2,
           ln1_g, ln1_b, ln2_g, ln2_b, lnf_g, lnf_b, Wc, bc):
    ph = IMG // PATCH
    p = x.reshape(NB, 3, ph, PATCH, ph, PATCH).transpose(
        0, 2, 4, 1, 3, 5).reshape(NB * ph * ph, 3 * PATCH * PATCH)
    pe = pl.pallas_call(
        _pe_k,
        out_shape=jax.ShapeDtypeStruct((NB * ph * ph, D), f32),
    )(p, Wp, bp.reshape(1, D))
    h = jnp.concatenate(
        [jnp.broadcast_to(cls_tok, (NB, 1, D)), pe.reshape(NB, ph * ph, D)],
        axis=1) + pos

    full = lambda shape: pl.BlockSpec(shape, lambda e, c: (0,) * len(shape))
    in_specs = [
        full((NB, S, D)),
        full((1, D)), full((1, D)),
        full((D, 3 * D)), full((1, 3 * D)),
        full((D, D)), full((1, D)),
        full((1, D)), full((1, D)), full((D, NE)),
        pl.BlockSpec((1, D, F2), lambda e, c: (e, 0, c)),
        pl.BlockSpec((1, 1, F2), lambda e, c: (e, 0, c)),
        pl.BlockSpec((1, F2, D), lambda e, c: (e, c, 0)),
        pl.BlockSpec((1, 1, D), lambda e, c: (e, 0, 0)),
    ]
    scratch = [
        pltpu.VMEM((TT, D), bf16),
        pltpu.VMEM((NE, TT, CAP), bf16),
    ]
    b1r = b1.reshape(NE, 1, F)
    b2r = b2.reshape(NE, 1, D)

    flat, W1b, W2b = pl.pallas_call(
        _layer_store_k,
        grid=(NE, 2),
        in_specs=in_specs,
        out_specs=[
            pl.BlockSpec((TT, D), lambda e, c: (0, 0)),
            pl.BlockSpec((1, D, F2), lambda e, c: (e, 0, c)),
            pl.BlockSpec((1, F2, D), lambda e, c: (e, c, 0)),
        ],
        out_shape=[jax.ShapeDtypeStruct((TT, D), f32),
                   jax.ShapeDtypeStruct((NE, D, F), bf16),
                   jax.ShapeDtypeStruct((NE, F, D), bf16)],
        scratch_shapes=scratch,
    )(h, ln1_g[0].reshape(1, D), ln1_b[0].reshape(1, D),
      Wqkv, bqkv.reshape(1, 3 * D), Wo, bo.reshape(1, D),
      ln2_g[0].reshape(1, D), ln2_b[0].reshape(1, D), Wg,
      W1, b1r, W2, b2r)
    h = flat.reshape(NB, S, D)

    flat = pl.pallas_call(
        _layer_plain_k,
        grid=(NE, 2),
        in_specs=in_specs,
        out_specs=pl.BlockSpec((TT, D), lambda e, c: (0, 0)),
        out_shape=jax.ShapeDtypeStruct((TT, D), f32),
        scratch_shapes=scratch,
    )(h, ln1_g[1].reshape(1, D), ln1_b[1].reshape(1, D),
      Wqkv, bqkv.reshape(1, 3 * D), Wo, bo.reshape(1, D),
      ln2_g[1].reshape(1, D), ln2_b[1].reshape(1, D), Wg,
      W1b, b1r, W2b, b2r)
    h = flat.reshape(NB, S, D)

    out = pl.pallas_call(
        _final_k,
        out_shape=jax.ShapeDtypeStruct((NB, NC), f32),
    )(h.reshape(TT, D), lnf_g.reshape(1, D), lnf_b.reshape(1, D),
      Wc, bc.reshape(1, NC))
    return out


# speedup vs baseline: 1.0974x; 1.0974x over previous
import jax
import jax.numpy as jnp
from jax.experimental import pallas as pl
from jax.experimental.pallas import tpu as pltpu

D = 768
NH = 12
DK = 64
F = 3072
F2 = F // 2
NE = 16
PATCH = 16
IMG = 224
NC = 1000
NB = 4
NP = (IMG // PATCH) ** 2
S = NP + 1
TT = NB * S
CAP = int(2 * 2.0 * TT / NE)
f32 = jnp.float32
bf16 = jnp.bfloat16


def _bdot(a, b):
    return jnp.dot(a.astype(bf16), b.astype(bf16), preferred_element_type=f32)


def _ln(x, g, b):
    mu = jnp.mean(x, axis=-1, keepdims=True)
    var = jnp.mean(jnp.square(x - mu), axis=-1, keepdims=True)
    return (x - mu) / jnp.sqrt(var + 1e-6) * g + b


def _attn_block(x, g_ref, b_ref, wqkv_ref, bqkv_ref, wo_ref, bo_ref):
    xn = _ln(x, g_ref[...], b_ref[...])
    qkv = _bdot(xn, wqkv_ref[...]) + bqkv_ref[...]
    heads = []
    for hh in range(NH):
        q = qkv[:, hh * DK:(hh + 1) * DK]
        k = qkv[:, D + hh * DK:D + (hh + 1) * DK]
        v = qkv[:, 2 * D + hh * DK:2 * D + (hh + 1) * DK]
        s = _bdot(q, k.T) * (1.0 / 8.0)
        p = jax.nn.softmax(s, axis=-1)
        heads.append(_bdot(p, v))
    o = jnp.concatenate(heads, axis=-1)
    return x + _bdot(o, wo_ref[...]) + bo_ref[...]


def _route_body(x_ref, g_ref, b_ref, wg_ref, xn_s, cmb_s):
    x = x_ref[...]
    xn = _ln(x, g_ref[...], b_ref[...])
    xn_s[...] = xn.astype(bf16)
    logits = jnp.dot(xn, wg_ref[...], preferred_element_type=f32)
    gates = jax.nn.softmax(logits, axis=-1)
    ei = jax.lax.broadcasted_iota(jnp.int32, (TT, NE), 1)
    mx1 = jnp.max(gates, axis=-1, keepdims=True)
    i1 = jnp.min(jnp.where(gates == mx1, ei, NE), axis=-1, keepdims=True)
    m1 = (ei == i1).astype(f32)
    gm = gates * (1.0 - m1)
    mx2 = jnp.max(gm, axis=-1, keepdims=True)
    i2 = jnp.min(jnp.where(gm == mx2, ei, NE), axis=-1, keepdims=True)
    m2 = (ei == i2).astype(f32)
    rt = jax.lax.broadcasted_iota(jnp.int32, (TT, TT), 0)
    ct = jax.lax.broadcasted_iota(jnp.int32, (TT, TT), 1)
    tri = (ct < rt).astype(f32)
    pos1 = jnp.dot(tri, m1, preferred_element_type=f32)
    pos2 = jnp.dot(tri, m2, preferred_element_type=f32) + jnp.sum(
        m1, axis=0, keepdims=True)
    m1c = m1 * (pos1 < CAP)
    m2c = m2 * (pos2 < CAP)
    g1 = jnp.sum(gates * m1c, axis=-1, keepdims=True)
    g2 = jnp.sum(gates * m2c, axis=-1, keepdims=True)
    den = g1 + g2 + 1e-9
    g1 = g1 / den
    g2 = g2 / den
    loc1 = jnp.sum(pos1 * m1c, axis=-1, keepdims=True).astype(jnp.int32)
    loc2 = jnp.sum(pos2 * m2c, axis=-1, keepdims=True).astype(jnp.int32)
    ci = jax.lax.broadcasted_iota(jnp.int32, (TT, CAP), 1)
    oh1 = (ci == loc1).astype(f32) * g1
    oh2 = (ci == loc2).astype(f32) * g2
    for e in range(NE):
        cmb_s[e] = (m1c[:, e:e + 1] * oh1
                    + m2c[:, e:e + 1] * oh2).astype(bf16)


def _ffn_step(e, ch, w1_ref, b1_ref, w2_ref, b2_ref, o_ref, xn_s, cmb_s):
    c = cmb_s[e].astype(f32)
    dm = (c > 0.0).astype(bf16)
    ein = jax.lax.dot_general(dm, xn_s[...],
                              (((0,), (0,)), ((), ())),
                              preferred_element_type=f32)
    hh = _bdot(ein, w1_ref[0]) + b1_ref[0]
    hh = jax.nn.gelu(hh)
    eo = _bdot(hh, w2_ref[0])
    eo = eo + jnp.where(ch == 0, 1.0, 0.0) * b2_ref[0]
    o_ref[...] = o_ref[...] + _bdot(c, eo)


def _layer1_k(p_ref, cls_ref, posr_ref, wp_ref, bp_ref,
              g1_ref, bg1_ref, wqkv_ref, bqkv_ref, wo_ref, bo_ref,
              g2_ref, bg2_ref, wg_ref, w1_ref, b1_ref, w2_ref, b2_ref,
              o_ref, xn_s, cmb_s):
    e = pl.program_id(0)
    ch = pl.program_id(1)

    @pl.when((e == 0) & (ch == 0))
    def _prologue():
        pemb = _bdot(p_ref[...], wp_ref[...]) + bp_ref[...]
        for b in range(NB):
            xb = jnp.concatenate(
                [cls_ref[...], pemb[b * NP:(b + 1) * NP] + posr_ref[...]],
                axis=0)
            o_ref[pl.ds(b * S, S), :] = _attn_block(
                xb, g1_ref, bg1_ref, wqkv_ref, bqkv_ref, wo_ref, bo_ref)
        _route_body(o_ref, g2_ref, bg2_ref, wg_ref, xn_s, cmb_s)

    _ffn_step(e, ch, w1_ref, b1_ref, w2_ref, b2_ref, o_ref, xn_s, cmb_s)


def _layer2_k(h_ref, g1_ref, bg1_ref, wqkv_ref, bqkv_ref, wo_ref, bo_ref,
              g2_ref, bg2_ref, wg_ref, w1_ref, b1_ref, w2_ref, b2_ref,
              gf_ref, bf_ref, wc_ref, bc_ref,
              o_ref, out2_ref, xn_s, cmb_s):
    e = pl.program_id(0)
    ch = pl.program_id(1)

    @pl.when((e == 0) & (ch == 0))
    def _prologue():
        for b in range(NB):
            xb = h_ref[pl.ds(b * S, S), :]
            o_ref[pl.ds(b * S, S), :] = _attn_block(
                xb, g1_ref, bg1_ref, wqkv_ref, bqkv_ref, wo_ref, bo_ref)
        _route_body(o_ref, g2_ref, bg2_ref, wg_ref, xn_s, cmb_s)

    _ffn_step(e, ch, w1_ref, b1_ref, w2_ref, b2_ref, o_ref, xn_s, cmb_s)

    @pl.when((e == NE - 1) & (ch == 1))
    def _epilogue():
        xn = _ln(o_ref[...], gf_ref[...], bf_ref[...])
        bi = jax.lax.broadcasted_iota(jnp.int32, (NB, TT), 0)
        ti = jax.lax.broadcasted_iota(jnp.int32, (NB, TT), 1)
        pool = ((ti >= bi * S) & (ti < bi * S + S)).astype(f32) * (1.0 / S)
        pooled = jnp.dot(pool, xn, preferred_element_type=f32)
        out2_ref[...] = _bdot(pooled, wc_ref[...]) + bc_ref[...]


def kernel(x, Wp, bp, cls_tok, pos, Wqkv, bqkv, Wo, bo, Wg, W1, b1, W2, b2,
           ln1_g, ln1_b, ln2_g, ln2_b, lnf_g, lnf_b, Wc, bc):
    p = x.reshape(NB, 3, IMG // PATCH, PATCH, IMG // PATCH, PATCH).transpose(
        0, 2, 4, 1, 3, 5).reshape(NB * NP, 3 * PATCH * PATCH)
    clsrow = cls_tok.reshape(1, D) + pos[0, 0].reshape(1, D)
    posr = pos[0, 1:]

    full = lambda shape: pl.BlockSpec(shape, lambda e, c: (0,) * len(shape))
    wspecs = [
        pl.BlockSpec((1, D, F2), lambda e, c: (e, 0, c)),
        pl.BlockSpec((1, 1, F2), lambda e, c: (e, 0, c)),
        pl.BlockSpec((1, F2, D), lambda e, c: (e, c, 0)),
        pl.BlockSpec((1, 1, D), lambda e, c: (e, 0, 0)),
    ]
    attn_specs = [
        full((1, D)), full((1, D)),
        full((D, 3 * D)), full((1, 3 * D)),
        full((D, D)), full((1, D)),
        full((1, D)), full((1, D)), full((D, NE)),
    ]
    scratch = [
        pltpu.VMEM((TT, D), bf16),
        pltpu.VMEM((NE, TT, CAP), bf16),
    ]
    b1r = b1.reshape(NE, 1, F)
    b2r = b2.reshape(NE, 1, D)

    flat = pl.pallas_call(
        _layer1_k,
        grid=(NE, 2),
        in_specs=[full((NB * NP, 3 * PATCH * PATCH)), full((1, D)),
                  full((NP, D)), full((3 * PATCH * PATCH, D)), full((1, D))]
                 + attn_specs + wspecs,
        out_specs=pl.BlockSpec((TT, D), lambda e, c: (0, 0)),
        out_shape=jax.ShapeDtypeStruct((TT, D), f32),
        scratch_shapes=scratch,
    )(p, clsrow, posr, Wp, bp.reshape(1, D),
      ln1_g[0].reshape(1, D), ln1_b[0].reshape(1, D),
      Wqkv, bqkv.reshape(1, 3 * D), Wo, bo.reshape(1, D),
      ln2_g[0].reshape(1, D), ln2_b[0].reshape(1, D), Wg,
      W1, b1r, W2, b2r)

    _, out = pl.pallas_call(
        _layer2_k,
        grid=(NE, 2),
        in_specs=[full((TT, D))] + attn_specs + wspecs
                 + [full((1, D)), full((1, D)), full((D, NC)), full((1, NC))],
        out_specs=[pl.BlockSpec((TT, D), lambda e, c: (0, 0)),
                   full((NB, NC))],
        out_shape=[jax.ShapeDtypeStruct((TT, D), f32),
                   jax.ShapeDtypeStruct((NB, NC), f32)],
        scratch_shapes=scratch,
    )(flat,
      ln1_g[1].reshape(1, D), ln1_b[1].reshape(1, D),
      Wqkv, bqkv.reshape(1, 3 * D), Wo, bo.reshape(1, D),
      ln2_g[1].reshape(1, D), ln2_b[1].reshape(1, D), Wg,
      W1, b1r, W2, b2r,
      lnf_g.reshape(1, D), lnf_b.reshape(1, D), Wc, bc.reshape(1, NC))
    return out
